# Initial kernel scaffold; baseline (speedup 1.0000x reference)
import jax
import jax.numpy as jnp
from jax.experimental import pallas as pl


def kernel(x, edge_index, edge_attr, W_edge, W_upd, W_mlp):
    raise NotImplementedError("write your pallas kernel here")


# speedup vs baseline: 1.7465x; 1.7465x over previous
import functools

import jax
import jax.numpy as jnp
from jax import lax
from jax.experimental import pallas as pl
from jax.experimental.pallas import tpu as pltpu
from jax.experimental.pallas import tpu_sc as plsc

N_NODES = 10000
N_EDGES = 320000
D_FEAT = 128
D_EDGE = 16
N_LAYERS = 2
MP_SCALING = 0.1

NC = 2
NS = 16
LANES = 16
NW = NC * NS

C = 128
KCH = 80
GC = 8
EPAD = NW * KCH * C

NPAD = 10240
NPW = NPAD // NS


def _sc_layer_body(h_hbm, src_hbm, dst_hbm, gate_hbm, out_hbm,
                   src_v, dst_v, rows_v, gate_v, agg_sh, sem):
    c = lax.axis_index("c")
    s = lax.axis_index("s")
    wid = c * NS + s

    @pl.loop(0, C)
    def _zero_rows(e):
        for j in range(D_FEAT // LANES):
            rows_v[e, pl.ds(j * LANES, LANES)] = jnp.zeros((LANES,), jnp.float32)

    for t in range(NPW // C):
        pltpu.sync_copy(rows_v, agg_sh.at[pl.ds(s * NPW + t * C, C)])

    plsc.subcore_barrier()

    @pl.loop(0, KCH // GC)
    def _group(g):
        gbase = wid * KCH + g * GC
        pltpu.sync_copy(src_hbm.at[pl.ds(gbase, GC)], src_v)
        pltpu.sync_copy(dst_hbm.at[pl.ds(gbase, GC)], dst_v)

        @pl.loop(0, GC)
        def _chunk(k):
            base = (gbase + k) * C
            pltpu.async_copy(h_hbm.at[src_v.at[k]], rows_v, sem).wait()
            pltpu.sync_copy(gate_hbm.at[pl.ds(base, C)], gate_v)

            @pl.loop(0, C)
            def _edge(e):
                for j in range(D_FEAT // LANES):
                    sl = pl.ds(j * LANES, LANES)
                    rows_v[e, sl] = rows_v[e, sl] * gate_v[e, sl]

            pltpu.sync_copy(rows_v, agg_sh.at[dst_v.at[k]], add=True)

    plsc.subcore_barrier()

    for t in range(NPW // C):
        pltpu.sync_copy(agg_sh.at[pl.ds(s * NPW + t * C, C)], rows_v)
        pltpu.sync_copy(rows_v, out_hbm.at[c, pl.ds(s * NPW + t * C, C)])


_sc_layer = functools.partial(
    pl.kernel,
    out_type=jax.ShapeDtypeStruct((NC, NPAD, D_FEAT), jnp.float32),
    mesh=plsc.VectorSubcoreMesh(core_axis_name="c", subcore_axis_name="s"),
    scratch_types=[
        pltpu.VMEM((GC, C), jnp.int32),
        pltpu.VMEM((GC, C), jnp.int32),
        pltpu.VMEM((C, D_FEAT), jnp.float32),
        pltpu.VMEM((C, D_FEAT), jnp.float32),
        pltpu.VMEM_SHARED((NPAD, D_FEAT), jnp.float32),
        pltpu.SemaphoreType.DMA,
    ],
)(_sc_layer_body)


def _gate_body(ea_ref, w_ref, out_ref):
    out_ref[...] = jnp.dot(ea_ref[...], w_ref[...],
                           preferred_element_type=jnp.float32)


def _upd_body(p_ref, h_ref, w_ref, out_ref):
    agg = p_ref[0] + p_ref[1]
    a = MP_SCALING * jnp.dot(agg, w_ref[...],
                             preferred_element_type=jnp.float32) + h_ref[...]
    out_ref[...] = a * jax.nn.sigmoid(a)


def _fin_body(p_ref, h_ref, w1_ref, w2_ref, out_ref):
    agg = p_ref[0] + p_ref[1]
    a = MP_SCALING * jnp.dot(agg, w1_ref[...],
                             preferred_element_type=jnp.float32) + h_ref[...]
    h2 = a * jax.nn.sigmoid(a)
    b = jnp.dot(h2, w2_ref[...], preferred_element_type=jnp.float32)
    out_ref[...] = b * jax.nn.sigmoid(b)


_GATE_R = 1024
_ROW_R = 400


def kernel(x, edge_index, edge_attr, W_edge, W_upd, W_mlp):
    pad = EPAD - N_EDGES
    src = jnp.concatenate([edge_index[0], jnp.zeros((pad,), jnp.int32)])
    dst = jnp.concatenate([edge_index[1], jnp.zeros((pad,), jnp.int32)])
    src2 = src.reshape(NW * KCH, C)
    dst2 = dst.reshape(NW * KCH, C)
    ea_pad = jnp.concatenate([edge_attr, jnp.zeros((pad, D_EDGE), jnp.float32)])

    gate = pl.pallas_call(
        _gate_body,
        grid=(EPAD // _GATE_R,),
        in_specs=[pl.BlockSpec((_GATE_R, D_EDGE), lambda i: (i, 0)),
                  pl.BlockSpec((D_EDGE, D_FEAT), lambda i: (0, 0))],
        out_specs=pl.BlockSpec((_GATE_R, D_FEAT), lambda i: (i, 0)),
        out_shape=jax.ShapeDtypeStruct((EPAD, D_FEAT), jnp.float32),
    )(ea_pad, W_edge)

    h = x
    for l in range(N_LAYERS):
        parts = _sc_layer(h, src2, dst2, gate)
        if l < N_LAYERS - 1:
            h = pl.pallas_call(
                _upd_body,
                grid=(N_NODES // _ROW_R,),
                in_specs=[
                    pl.BlockSpec((NC, _ROW_R, D_FEAT), lambda i: (0, i, 0)),
                    pl.BlockSpec((_ROW_R, D_FEAT), lambda i: (i, 0)),
                    pl.BlockSpec((D_FEAT, D_FEAT), lambda i: (0, 0)),
                ],
                out_specs=pl.BlockSpec((_ROW_R, D_FEAT), lambda i: (i, 0)),
                out_shape=jax.ShapeDtypeStruct((N_NODES, D_FEAT), jnp.float32),
            )(parts, h, W_upd[l])
        else:
            out = pl.pallas_call(
                _fin_body,
                grid=(N_NODES // _ROW_R,),
                in_specs=[
                    pl.BlockSpec((NC, _ROW_R, D_FEAT), lambda i: (0, i, 0)),
                    pl.BlockSpec((_ROW_R, D_FEAT), lambda i: (i, 0)),
                    pl.BlockSpec((D_FEAT, D_FEAT), lambda i: (0, 0)),
                    pl.BlockSpec((D_FEAT, D_FEAT), lambda i: (0, 0)),
                ],
                out_specs=pl.BlockSpec((_ROW_R, D_FEAT), lambda i: (i, 0)),
                out_shape=jax.ShapeDtypeStruct((N_NODES, D_FEAT), jnp.float32),
            )(parts, h, W_upd[l], W_mlp)
    return out


# speedup vs baseline: 2.1183x; 1.2129x over previous
import functools

import jax
import jax.numpy as jnp
from jax import lax
from jax.experimental import pallas as pl
from jax.experimental.pallas import tpu as pltpu
from jax.experimental.pallas import tpu_sc as plsc

N_NODES = 10000
N_EDGES = 320000
D_FEAT = 128
D_EDGE = 16
N_LAYERS = 2
MP_SCALING = 0.1

NC = 2
NS = 16
LANES = 16
NW = NC * NS

C = 128
KCH = 80
GC = 4
EPAD = NW * KCH * C

SLAB = 632
SLAB_LAST = N_NODES - SLAB * (NS - 1)


def _sc_layer_body(h_hbm, src_hbm, dst_hbm, gate_hbm, out_hbm,
                   src_v, dst_v, rows_v, gate_v, agg_sh,
                   sem_g0, sem_g1, sem_t, sem_s0, sem_s1):
    c = lax.axis_index("c")
    s = lax.axis_index("s")
    wid = c * NS + s
    sem_g = (sem_g0, sem_g1)
    sem_s = (sem_s0, sem_s1)
    slab = s * SLAB

    @pl.loop(0, C)
    def _zero_rows(e):
        for j in range(D_FEAT // LANES):
            rows_v[0, e, pl.ds(j * LANES, LANES)] = jnp.zeros((LANES,),
                                                              jnp.float32)

    def _slab_copy(move):
        for t in range(4):
            move(pl.ds(slab + t * C, C), C)

        @pl.when(s < NS - 1)
        def _tail_full():
            move(pl.ds(slab + 4 * C, SLAB - 4 * C), SLAB - 4 * C)

        @pl.when(s == NS - 1)
        def _tail_last():
            move(pl.ds(slab + 4 * C, SLAB_LAST - 4 * C), SLAB_LAST - 4 * C)

    _slab_copy(lambda sl, n: pltpu.sync_copy(rows_v.at[0, pl.ds(0, n)],
                                             agg_sh.at[sl]))
    plsc.subcore_barrier()

    @pl.loop(0, KCH // GC)
    def _group(g):
        gbase = wid * KCH + g * GC
        pltpu.sync_copy(src_hbm.at[pl.ds(gbase, GC)], src_v)
        pltpu.sync_copy(dst_hbm.at[pl.ds(gbase, GC)], dst_v)

        def _gather(k):
            return pltpu.async_copy(h_hbm.at[src_v.at[k]],
                                    rows_v.at[k % 2], sem_g[k % 2])

        def _gate_load(k):
            return pltpu.async_copy(gate_hbm.at[pl.ds((gbase + k) * C, C)],
                                    gate_v, sem_t)

        gath = [None, None]
        scat = [None, None]
        gath[0] = _gather(0)
        gl = _gate_load(0)
        for k in range(GC):
            b = k % 2
            nb = (k + 1) % 2
            if k + 1 < GC:
                if scat[nb] is not None:
                    scat[nb].wait()
                    scat[nb] = None
                gath[nb] = _gather(k + 1)
            gath[b].wait()
            gl.wait()

            @pl.loop(0, C)
            def _edge(e):
                for j in range(D_FEAT // LANES):
                    sl = pl.ds(j * LANES, LANES)
                    rows_v[b, e, sl] = rows_v[b, e, sl] * gate_v[e, sl]

            if k + 1 < GC:
                gl = _gate_load(k + 1)
            scat[b] = pltpu.async_copy(rows_v.at[b], agg_sh.at[dst_v.at[k]],
                                       sem_s[b], add=True)
        for b in range(2):
            if scat[b] is not None:
                scat[b].wait()

    plsc.subcore_barrier()

    def _out_move(sl, n):
        pltpu.sync_copy(agg_sh.at[sl], rows_v.at[0, pl.ds(0, n)])
        pltpu.sync_copy(rows_v.at[0, pl.ds(0, n)], out_hbm.at[c, sl])

    _out_move(pl.ds(slab, C), C)
    for t in range(1, 4):
        _out_move(pl.ds(slab + t * C, C), C)

    @pl.when(s < NS - 1)
    def _tail_full():
        _out_move(pl.ds(slab + 4 * C, SLAB - 4 * C), SLAB - 4 * C)

    @pl.when(s == NS - 1)
    def _tail_last():
        _out_move(pl.ds(slab + 4 * C, SLAB_LAST - 4 * C), SLAB_LAST - 4 * C)


_sc_layer = functools.partial(
    pl.kernel,
    out_type=jax.ShapeDtypeStruct((NC, N_NODES, D_FEAT), jnp.float32),
    mesh=plsc.VectorSubcoreMesh(core_axis_name="c", subcore_axis_name="s"),
    scratch_types=[
        pltpu.VMEM((GC, C), jnp.int32),
        pltpu.VMEM((GC, C), jnp.int32),
        pltpu.VMEM((2, C, D_FEAT), jnp.float32),
        pltpu.VMEM((C, D_FEAT), jnp.float32),
        pltpu.VMEM_SHARED((N_NODES, D_FEAT), jnp.float32),
        pltpu.SemaphoreType.DMA,
        pltpu.SemaphoreType.DMA,
        pltpu.SemaphoreType.DMA,
        pltpu.SemaphoreType.DMA,
        pltpu.SemaphoreType.DMA,
    ],
)(_sc_layer_body)


def _gate_body(ea_ref, w_ref, out_ref):
    out_ref[...] = jnp.dot(ea_ref[...], w_ref[...],
                           preferred_element_type=jnp.float32)


def _upd_body(p_ref, h_ref, w_ref, out_ref):
    agg = p_ref[0] + p_ref[1]
    a = MP_SCALING * jnp.dot(agg, w_ref[...],
                             preferred_element_type=jnp.float32) + h_ref[...]
    out_ref[...] = a * jax.nn.sigmoid(a)


def _fin_body(p_ref, h_ref, w1_ref, w2_ref, out_ref):
    agg = p_ref[0] + p_ref[1]
    a = MP_SCALING * jnp.dot(agg, w1_ref[...],
                             preferred_element_type=jnp.float32) + h_ref[...]
    h2 = a * jax.nn.sigmoid(a)
    b = jnp.dot(h2, w2_ref[...], preferred_element_type=jnp.float32)
    out_ref[...] = b * jax.nn.sigmoid(b)


_GATE_R = 1024
_ROW_R = 400


def kernel(x, edge_index, edge_attr, W_edge, W_upd, W_mlp):
    pad = EPAD - N_EDGES
    src = jnp.concatenate([edge_index[0], jnp.zeros((pad,), jnp.int32)])
    dst = jnp.concatenate([edge_index[1], jnp.zeros((pad,), jnp.int32)])
    src2 = src.reshape(NW * KCH, C)
    dst2 = dst.reshape(NW * KCH, C)
    ea_pad = jnp.concatenate([edge_attr, jnp.zeros((pad, D_EDGE), jnp.float32)])

    gate = pl.pallas_call(
        _gate_body,
        grid=(EPAD // _GATE_R,),
        in_specs=[pl.BlockSpec((_GATE_R, D_EDGE), lambda i: (i, 0)),
                  pl.BlockSpec((D_EDGE, D_FEAT), lambda i: (0, 0))],
        out_specs=pl.BlockSpec((_GATE_R, D_FEAT), lambda i: (i, 0)),
        out_shape=jax.ShapeDtypeStruct((EPAD, D_FEAT), jnp.float32),
    )(ea_pad, W_edge)

    h = x
    for l in range(N_LAYERS):
        parts = _sc_layer(h, src2, dst2, gate)
        if l < N_LAYERS - 1:
            h = pl.pallas_call(
                _upd_body,
                grid=(N_NODES // _ROW_R,),
                in_specs=[
                    pl.BlockSpec((NC, _ROW_R, D_FEAT), lambda i: (0, i, 0)),
                    pl.BlockSpec((_ROW_R, D_FEAT), lambda i: (i, 0)),
                    pl.BlockSpec((D_FEAT, D_FEAT), lambda i: (0, 0)),
                ],
                out_specs=pl.BlockSpec((_ROW_R, D_FEAT), lambda i: (i, 0)),
                out_shape=jax.ShapeDtypeStruct((N_NODES, D_FEAT), jnp.float32),
            )(parts, h, W_upd[l])
        else:
            out = pl.pallas_call(
                _fin_body,
                grid=(N_NODES // _ROW_R,),
                in_specs=[
                    pl.BlockSpec((NC, _ROW_R, D_FEAT), lambda i: (0, i, 0)),
                    pl.BlockSpec((_ROW_R, D_FEAT), lambda i: (i, 0)),
                    pl.BlockSpec((D_FEAT, D_FEAT), lambda i: (0, 0)),
                    pl.BlockSpec((D_FEAT, D_FEAT), lambda i: (0, 0)),
                ],
                out_specs=pl.BlockSpec((_ROW_R, D_FEAT), lambda i: (i, 0)),
                out_shape=jax.ShapeDtypeStruct((N_NODES, D_FEAT), jnp.float32),
            )(parts, h, W_upd[l], W_mlp)
    return out
